# baseline (speedup 1.0000x reference)
import jax
import jax.numpy as jnp
from jax.experimental import pallas as pl
from jax.experimental.pallas import tpu as pltpu

B, SEQ, EMB = 16, 128, 128
HEADS = 4
HD = EMB // HEADS
LAT = 64
E = 16
TOPK = 2
FF = 512
BASE = 10000.0
T = B * SEQ

KTILE = 128
KSEQ = KTILE // EMB
NK = (SEQ * EMB) // KTILE
NSTEPS = NK + 1 + E


def _rmsnorm_rows(x, g, eps=1e-5):
    return x * jax.lax.rsqrt(jnp.mean(x * x, axis=-1, keepdims=True) + eps) * g


def _block_kernel(x4_ref, x_ref, g1_ref, g2_ref, wl_ref, bl_ref,
                  wq_ref, bq_ref, wk_ref, bk_ref, wv_ref, bv_ref,
                  wo_ref, bo_ref, wr_ref, br_ref,
                  w1_ref, b1_ref, ws_ref, bs_ref, w2_ref, b2_ref,
                  out_ref, lat_s, h_s, hn_s, gates_s):
    i = pl.program_id(0)

    @pl.when(i < NK)
    def _():
        xn = _rmsnorm_rows(x4_ref[:, i], g1_ref[...])
        xf = xn.reshape(B, KTILE)
        acc = jax.lax.dot_general(
            xf, wl_ref[...], (((1,), (0,)), ((), ())),
            preferred_element_type=jnp.float32)

        @pl.when(i == 0)
        def _():
            lat_s[...] = acc + bl_ref[...]

        @pl.when(i > 0)
        def _():
            lat_s[...] += acc

    @pl.when(i == NK)
    def _():
        xn = _rmsnorm_rows(x_ref[...], g1_ref[...])
        lat = lat_s[...].reshape(B, LAT, EMB)

        xnb = xn.astype(jnp.bfloat16)
        latb = lat.astype(jnp.bfloat16)
        q = jax.lax.dot_general(xnb, wq_ref[...].astype(jnp.bfloat16),
                                (((2,), (0,)), ((), ())),
                                preferred_element_type=jnp.float32) + bq_ref[...]
        k = jax.lax.dot_general(latb, wk_ref[...].astype(jnp.bfloat16),
                                (((2,), (0,)), ((), ())),
                                preferred_element_type=jnp.float32) + bk_ref[...]
        v = jax.lax.dot_general(latb, wv_ref[...].astype(jnp.bfloat16),
                                (((2,), (0,)), ((), ())),
                                preferred_element_type=jnp.float32) + bv_ref[...]

        lane = jax.lax.broadcasted_iota(jnp.int32, (SEQ, EMB), 1)
        pos = jax.lax.broadcasted_iota(jnp.int32, (SEQ, EMB), 0).astype(
            jnp.float32)
        pair = (lane % HD) // 2
        inv_freq = jnp.exp(pair.astype(jnp.float32) * (-2.0 / HD)
                           * jnp.log(BASE))
        ang = pos * inv_freq
        c = jnp.cos(ang)[None]
        s = jnp.sin(ang)[None]
        even = (lane % 2) == 0
        q_nxt = jnp.roll(q, -1, axis=2)
        q_prv = jnp.roll(q, 1, axis=2)
        q = jnp.where(even[None], q * c - q_nxt * s, q_prv * s + q * c)

        scale = 1.0 / (HD ** 0.5)
        o_heads = []
        for hd in range(HEADS):
            qh = q[:, :, hd * HD:(hd + 1) * HD]
            kh = k[:, :, hd * HD:(hd + 1) * HD]
            vh = v[:, :, hd * HD:(hd + 1) * HD]
            sc = jax.lax.dot_general(qh, kh, (((2,), (2,)), ((0,), (0,))),
                                     preferred_element_type=jnp.float32) * scale
            sc = sc - jnp.max(sc, axis=-1, keepdims=True)
            w = jnp.exp(sc)
            w = w / jnp.sum(w, axis=-1, keepdims=True)
            o_heads.append(jax.lax.dot_general(
                w, vh, (((2,), (1,)), ((0,), (0,))),
                preferred_element_type=jnp.float32))
        o = jnp.concatenate(o_heads, axis=2)

        h = x_ref[...] + jax.lax.dot_general(
            o.astype(jnp.bfloat16), wo_ref[...].astype(jnp.bfloat16),
            (((2,), (0,)), ((), ())),
            preferred_element_type=jnp.float32) + bo_ref[...]
        h2d = h.reshape(T, EMB)
        h_s[...] = h2d

        hn = _rmsnorm_rows(h2d, g2_ref[...])
        hn_s[...] = hn

        logits = jax.lax.dot_general(hn, wr_ref[...], (((1,), (0,)), ((), ())),
                                     preferred_element_type=jnp.float32
                                     ) + br_ref[...]
        logits = logits - jnp.max(logits, axis=-1, keepdims=True)
        p = jnp.exp(logits)
        p = p / jnp.sum(p, axis=-1, keepdims=True)

        eidx = jax.lax.broadcasted_iota(jnp.int32, (T, E), 1)
        p1 = jnp.max(p, axis=-1, keepdims=True)
        i1 = jnp.min(jnp.where(p == p1, eidx, E), axis=-1, keepdims=True)
        pm = jnp.where(eidx == i1, -jnp.inf, p)
        p2 = jnp.max(pm, axis=-1, keepdims=True)
        i2 = jnp.min(jnp.where(pm == p2, eidx, E), axis=-1, keepdims=True)
        gates_s[...] = (jnp.where(eidx == i1, p1, 0.0)
                        + jnp.where(eidx == i2, p2, 0.0))

    @pl.when(i > NK)
    def _():
        e = i - NK - 1
        hn = hn_s[...].astype(jnp.bfloat16)
        h1 = jax.lax.dot_general(hn, w1_ref[0].astype(jnp.bfloat16),
                                 (((1,), (0,)), ((), ())),
                                 preferred_element_type=jnp.float32) + b1_ref[0]
        h2 = jax.lax.dot_general(h1.astype(jnp.bfloat16),
                                 ws_ref[0].astype(jnp.bfloat16),
                                 (((1,), (0,)), ((), ())),
                                 preferred_element_type=jnp.float32) + bs_ref[0]
        h2 = jnp.maximum(h2, 0.0)
        eo = jax.lax.dot_general(h2.astype(jnp.bfloat16),
                                 w2_ref[0].astype(jnp.bfloat16),
                                 (((1,), (0,)), ((), ())),
                                 preferred_element_type=jnp.float32) + b2_ref[0]
        eidx = jax.lax.broadcasted_iota(jnp.int32, (T, E), 1)
        g = jnp.sum(jnp.where(eidx == e, gates_s[...], 0.0), axis=-1,
                    keepdims=True)

        @pl.when(e == 0)
        def _():
            out_ref[...] = h_s[...] + g * eo

        @pl.when(e > 0)
        def _():
            out_ref[...] += g * eo


def kernel(x, g1, g2, Wl, bl, Wq, bq, Wk, bk, Wv, bv, Wo, bo, Wr, br,
           W1, b1, Ws, bs, W2, b2):
    def eidx_map(i):
        return (jnp.clip(i - (NK + 1), 0, E - 1), 0, 0)

    out = pl.pallas_call(
        _block_kernel,
        grid=(NSTEPS,),
        in_specs=[
            pl.BlockSpec((B, NK, KSEQ, EMB), lambda i: (0, 0, 0, 0)),
            pl.BlockSpec((B, SEQ, EMB), lambda i: (0, 0, 0)),
            pl.BlockSpec((EMB,), lambda i: (0,)),
            pl.BlockSpec((EMB,), lambda i: (0,)),
            pl.BlockSpec((KTILE, LAT * EMB),
                         lambda i: (jnp.minimum(i, NK - 1), 0)),
            pl.BlockSpec((LAT * EMB,), lambda i: (0,)),
            pl.BlockSpec((EMB, EMB), lambda i: (0, 0)),
            pl.BlockSpec((EMB,), lambda i: (0,)),
            pl.BlockSpec((EMB, EMB), lambda i: (0, 0)),
            pl.BlockSpec((EMB,), lambda i: (0,)),
            pl.BlockSpec((EMB, EMB), lambda i: (0, 0)),
            pl.BlockSpec((EMB,), lambda i: (0,)),
            pl.BlockSpec((EMB, EMB), lambda i: (0, 0)),
            pl.BlockSpec((EMB,), lambda i: (0,)),
            pl.BlockSpec((EMB, E), lambda i: (0, 0)),
            pl.BlockSpec((E,), lambda i: (0,)),
            pl.BlockSpec((1, EMB, FF), eidx_map),
            pl.BlockSpec((1, 1, FF), eidx_map),
            pl.BlockSpec((1, FF, FF), eidx_map),
            pl.BlockSpec((1, 1, FF), eidx_map),
            pl.BlockSpec((1, FF, EMB), eidx_map),
            pl.BlockSpec((1, 1, EMB), eidx_map),
        ],
        out_specs=pl.BlockSpec((T, EMB), lambda i: (0, 0)),
        out_shape=jax.ShapeDtypeStruct((T, EMB), jnp.float32),
        scratch_shapes=[
            pltpu.VMEM((B, LAT * EMB), jnp.float32),
            pltpu.VMEM((T, EMB), jnp.float32),
            pltpu.VMEM((T, EMB), jnp.float32),
            pltpu.VMEM((T, E), jnp.float32),
        ],
    )(x.reshape(B, NK, KSEQ, EMB), x, g1, g2, Wl, bl,
      Wq, bq, Wk, bk, Wv, bv, Wo, bo, Wr, br,
      W1, b1.reshape(E, 1, FF), Ws, bs.reshape(E, 1, FF), W2,
      b2.reshape(E, 1, EMB))

    return out.reshape(B, SEQ, EMB)


# speedup vs baseline: 1.1334x; 1.1334x over previous
import jax
import jax.numpy as jnp
from jax.experimental import pallas as pl
from jax.experimental.pallas import tpu as pltpu

B, SEQ, EMB = 16, 128, 128
HEADS = 4
HD = EMB // HEADS
LAT = 64
E = 16
TOPK = 2
FF = 512
BASE = 10000.0
T = B * SEQ

KTILE = 256
KSEQ = KTILE // EMB
NK = (SEQ * EMB) // KTILE
NSTEPS = NK + 1 + E


def _rmsnorm_rows(x, g, eps=1e-5):
    return x * jax.lax.rsqrt(jnp.mean(x * x, axis=-1, keepdims=True) + eps) * g


def _block_kernel(x4_ref, x_ref, g1_ref, g2_ref, wl_ref, bl_ref,
                  wq_ref, bq_ref, wk_ref, bk_ref, wv_ref, bv_ref,
                  wo_ref, bo_ref, wr_ref, br_ref,
                  w1_ref, b1_ref, ws_ref, bs_ref, w2_ref, b2_ref,
                  out_ref, lat_s, h_s, hn_s, gates_s):
    i = pl.program_id(0)

    @pl.when(i < NK)
    def _():
        xn = _rmsnorm_rows(x4_ref[:, i], g1_ref[...])
        xf = xn.reshape(B, KTILE)
        acc = jax.lax.dot_general(
            xf, wl_ref[...], (((1,), (0,)), ((), ())),
            preferred_element_type=jnp.float32)

        @pl.when(i == 0)
        def _():
            lat_s[...] = acc + bl_ref[...]

        @pl.when(i > 0)
        def _():
            lat_s[...] += acc

    @pl.when(i == NK)
    def _():
        xn = _rmsnorm_rows(x_ref[...], g1_ref[...])
        lat = lat_s[...].reshape(B, LAT, EMB)

        xnb = xn.astype(jnp.bfloat16)
        latb = lat.astype(jnp.bfloat16)
        q = jax.lax.dot_general(xnb, wq_ref[...].astype(jnp.bfloat16),
                                (((2,), (0,)), ((), ())),
                                preferred_element_type=jnp.float32) + bq_ref[...]
        k = jax.lax.dot_general(latb, wk_ref[...].astype(jnp.bfloat16),
                                (((2,), (0,)), ((), ())),
                                preferred_element_type=jnp.float32) + bk_ref[...]
        v = jax.lax.dot_general(latb, wv_ref[...].astype(jnp.bfloat16),
                                (((2,), (0,)), ((), ())),
                                preferred_element_type=jnp.float32) + bv_ref[...]

        lane = jax.lax.broadcasted_iota(jnp.int32, (SEQ, EMB), 1)
        pos = jax.lax.broadcasted_iota(jnp.int32, (SEQ, EMB), 0).astype(
            jnp.float32)
        pair = (lane % HD) // 2
        inv_freq = jnp.exp(pair.astype(jnp.float32) * (-2.0 / HD)
                           * jnp.log(BASE))
        ang = pos * inv_freq
        c = jnp.cos(ang)[None]
        s = jnp.sin(ang)[None]
        even = (lane % 2) == 0
        q_nxt = jnp.roll(q, -1, axis=2)
        q_prv = jnp.roll(q, 1, axis=2)
        q = jnp.where(even[None], q * c - q_nxt * s, q_prv * s + q * c)

        scale = 1.0 / (HD ** 0.5)
        o_heads = []
        for hd in range(HEADS):
            qh = q[:, :, hd * HD:(hd + 1) * HD]
            kh = k[:, :, hd * HD:(hd + 1) * HD]
            vh = v[:, :, hd * HD:(hd + 1) * HD]
            sc = jax.lax.dot_general(qh, kh, (((2,), (2,)), ((0,), (0,))),
                                     preferred_element_type=jnp.float32) * scale
            sc = sc - jnp.max(sc, axis=-1, keepdims=True)
            w = jnp.exp(sc)
            w = w / jnp.sum(w, axis=-1, keepdims=True)
            o_heads.append(jax.lax.dot_general(
                w, vh, (((2,), (1,)), ((0,), (0,))),
                preferred_element_type=jnp.float32))
        o = jnp.concatenate(o_heads, axis=2)

        h = x_ref[...] + jax.lax.dot_general(
            o.astype(jnp.bfloat16), wo_ref[...].astype(jnp.bfloat16),
            (((2,), (0,)), ((), ())),
            preferred_element_type=jnp.float32) + bo_ref[...]
        h2d = h.reshape(T, EMB)
        h_s[...] = h2d

        hn = _rmsnorm_rows(h2d, g2_ref[...])
        hn_s[...] = hn

        logits = jax.lax.dot_general(hn, wr_ref[...], (((1,), (0,)), ((), ())),
                                     preferred_element_type=jnp.float32
                                     ) + br_ref[...]
        logits = logits - jnp.max(logits, axis=-1, keepdims=True)
        p = jnp.exp(logits)
        p = p / jnp.sum(p, axis=-1, keepdims=True)

        eidx = jax.lax.broadcasted_iota(jnp.int32, (T, E), 1)
        p1 = jnp.max(p, axis=-1, keepdims=True)
        i1 = jnp.min(jnp.where(p == p1, eidx, E), axis=-1, keepdims=True)
        pm = jnp.where(eidx == i1, -jnp.inf, p)
        p2 = jnp.max(pm, axis=-1, keepdims=True)
        i2 = jnp.min(jnp.where(pm == p2, eidx, E), axis=-1, keepdims=True)
        gates_s[...] = (jnp.where(eidx == i1, p1, 0.0)
                        + jnp.where(eidx == i2, p2, 0.0))

    @pl.when(i > NK)
    def _():
        e = i - NK - 1
        hn = hn_s[...].astype(jnp.bfloat16)
        h1 = jax.lax.dot_general(hn, w1_ref[0].astype(jnp.bfloat16),
                                 (((1,), (0,)), ((), ())),
                                 preferred_element_type=jnp.float32) + b1_ref[0]
        h2 = jax.lax.dot_general(h1.astype(jnp.bfloat16),
                                 ws_ref[0].astype(jnp.bfloat16),
                                 (((1,), (0,)), ((), ())),
                                 preferred_element_type=jnp.float32) + bs_ref[0]
        h2 = jnp.maximum(h2, 0.0)
        eo = jax.lax.dot_general(h2.astype(jnp.bfloat16),
                                 w2_ref[0].astype(jnp.bfloat16),
                                 (((1,), (0,)), ((), ())),
                                 preferred_element_type=jnp.float32) + b2_ref[0]
        eidx = jax.lax.broadcasted_iota(jnp.int32, (T, E), 1)
        g = jnp.sum(jnp.where(eidx == e, gates_s[...], 0.0), axis=-1,
                    keepdims=True)

        @pl.when(e == 0)
        def _():
            out_ref[...] = h_s[...] + g * eo

        @pl.when(e > 0)
        def _():
            out_ref[...] += g * eo


def kernel(x, g1, g2, Wl, bl, Wq, bq, Wk, bk, Wv, bv, Wo, bo, Wr, br,
           W1, b1, Ws, bs, W2, b2):
    def eidx_map(i):
        return (jnp.clip(i - (NK + 1), 0, E - 1), 0, 0)

    out = pl.pallas_call(
        _block_kernel,
        grid=(NSTEPS,),
        in_specs=[
            pl.BlockSpec((B, NK, KSEQ, EMB), lambda i: (0, 0, 0, 0)),
            pl.BlockSpec((B, SEQ, EMB), lambda i: (0, 0, 0)),
            pl.BlockSpec((EMB,), lambda i: (0,)),
            pl.BlockSpec((EMB,), lambda i: (0,)),
            pl.BlockSpec((KTILE, LAT * EMB),
                         lambda i: (jnp.minimum(i, NK - 1), 0)),
            pl.BlockSpec((LAT * EMB,), lambda i: (0,)),
            pl.BlockSpec((EMB, EMB), lambda i: (0, 0)),
            pl.BlockSpec((EMB,), lambda i: (0,)),
            pl.BlockSpec((EMB, EMB), lambda i: (0, 0)),
            pl.BlockSpec((EMB,), lambda i: (0,)),
            pl.BlockSpec((EMB, EMB), lambda i: (0, 0)),
            pl.BlockSpec((EMB,), lambda i: (0,)),
            pl.BlockSpec((EMB, EMB), lambda i: (0, 0)),
            pl.BlockSpec((EMB,), lambda i: (0,)),
            pl.BlockSpec((EMB, E), lambda i: (0, 0)),
            pl.BlockSpec((E,), lambda i: (0,)),
            pl.BlockSpec((1, EMB, FF), eidx_map),
            pl.BlockSpec((1, 1, FF), eidx_map),
            pl.BlockSpec((1, FF, FF), eidx_map),
            pl.BlockSpec((1, 1, FF), eidx_map),
            pl.BlockSpec((1, FF, EMB), eidx_map),
            pl.BlockSpec((1, 1, EMB), eidx_map),
        ],
        out_specs=pl.BlockSpec((T, EMB), lambda i: (0, 0)),
        out_shape=jax.ShapeDtypeStruct((T, EMB), jnp.float32),
        scratch_shapes=[
            pltpu.VMEM((B, LAT * EMB), jnp.float32),
            pltpu.VMEM((T, EMB), jnp.float32),
            pltpu.VMEM((T, EMB), jnp.float32),
            pltpu.VMEM((T, E), jnp.float32),
        ],
    )(x.reshape(B, NK, KSEQ, EMB), x, g1, g2, Wl, bl,
      Wq, bq, Wk, bk, Wv, bv, Wo, bo, Wr, br,
      W1, b1.reshape(E, 1, FF), Ws, bs.reshape(E, 1, FF), W2,
      b2.reshape(E, 1, EMB))

    return out.reshape(B, SEQ, EMB)
